# Initial kernel scaffold; baseline (speedup 1.0000x reference)
import jax
import jax.numpy as jnp
from jax.experimental import pallas as pl

_N = 4096
_R0, _C0, _R1, _C1 = 1024, 1024, 3071, 3071
_BR = 256


def _body(b_ref, bt_ref, o_ref):
    i = pl.program_id(0)
    r0 = i * _BR
    rows = jax.lax.broadcasted_iota(jnp.int32, (1, _BR, _N), 1) + r0
    cols = jax.lax.broadcasted_iota(jnp.int32, (1, _BR, _N), 2)
    mask = (rows >= _R0) & (rows <= _R1) & (cols >= _C0) & (cols <= _C1)
    o_ref[...] = jnp.where(mask, bt_ref[0, 0], b_ref[...])


def kernel(B, Bt):
    return pl.pallas_call(
        _body,
        grid=(_N // _BR,),
        in_specs=[
            pl.BlockSpec((1, _BR, _N), lambda i: (0, i, 0)),
            pl.BlockSpec(memory_space=pl.ANY),
        ],
        out_specs=pl.BlockSpec((1, _BR, _N), lambda i: (0, i, 0)),
        out_shape=jax.ShapeDtypeStruct((1, _N, _N), jnp.float32),
    )(B, Bt)


# speedup vs baseline: 437.2576x; 437.2576x over previous
import jax
import jax.numpy as jnp
from jax.experimental import pallas as pl

_N = 4096
_R0, _C0, _R1, _C1 = 1024, 1024, 3071, 3071
_BR = 256


def _body(b_ref, bt_ref, o_ref):
    i = pl.program_id(0)
    r0 = i * _BR
    rows = jax.lax.broadcasted_iota(jnp.int32, (1, _BR, _N), 1) + r0
    cols = jax.lax.broadcasted_iota(jnp.int32, (1, _BR, _N), 2)
    mask = (rows >= _R0) & (rows <= _R1) & (cols >= _C0) & (cols <= _C1)
    o_ref[...] = jnp.where(mask, bt_ref[0, 0], b_ref[...])


def kernel(B, Bt):
    return pl.pallas_call(
        _body,
        grid=(_N // _BR,),
        in_specs=[
            pl.BlockSpec((1, _BR, _N), lambda i: (0, i, 0)),
            pl.BlockSpec((1, 1), lambda i: (0, 0)),
        ],
        out_specs=pl.BlockSpec((1, _BR, _N), lambda i: (0, i, 0)),
        out_shape=jax.ShapeDtypeStruct((1, _N, _N), jnp.float32),
    )(B, Bt)
